# baseline (speedup 1.0000x reference)
import jax
import jax.numpy as jnp
from jax import lax
from jax.experimental import pallas as pl
from jax.experimental.pallas import tpu as pltpu
from jax.experimental.pallas import tpu_sc as plsc

_NC = 2
_NS = 16
_NW = _NC * _NS

_B = 4096
_L = 50
_E = 64
_TOT = _B * _L
_C = 128
_ROWS_PER_W = _TOT // _NW
_CH = _ROWS_PER_W // _C


def _emb_body(src_idx, tgt_idx, src_tab, tgt_tab, outs,
              idxs, idxt, buf0, buf1, s0, s1):
    wid = lax.axis_index("s") * _NC + lax.axis_index("c")
    row0 = wid * _ROWS_PER_W

    pltpu.sync_copy(src_idx.at[pl.ds(wid * _CH, _CH)], idxs)
    pltpu.sync_copy(tgt_idx.at[pl.ds(wid * _CH, _CH)], idxt)

    def run_table(tab, out, idxv):
        def gather(c, buf, sem):
            pltpu.async_copy(tab.at[idxv.at[c]], buf, sem)

        def out_block(c):
            return out.at[pl.ds(row0 + c * _C, _C)]

        gather(0, buf0, s0)

        @pl.loop(0, _CH, step=2)
        def _pair(c):
            gather(c + 1, buf1, s1)
            pltpu.make_async_copy(tab.at[idxv.at[c]], buf0, s0).wait()
            pltpu.sync_copy(buf0, out_block(c))

            @pl.when(c + 2 < _CH)
            def _():
                gather(c + 2, buf0, s0)

            pltpu.make_async_copy(tab.at[idxv.at[c + 1]], buf1, s1).wait()
            pltpu.sync_copy(buf1, out_block(c + 1))

    run_table(src_tab, outs.at[0], idxs)
    run_table(tgt_tab, outs.at[1], idxt)


@jax.jit
def _emb(src_idx2d, tgt_idx2d, src_table, tgt_table):
    mesh = plsc.VectorSubcoreMesh(core_axis_name="c", subcore_axis_name="s")
    out_type = jax.ShapeDtypeStruct((2, _TOT, _E), jnp.float32)
    scratch = [
        pltpu.VMEM((_CH, _C), jnp.int32),
        pltpu.VMEM((_CH, _C), jnp.int32),
        pltpu.VMEM((_C, _E), jnp.float32),
        pltpu.VMEM((_C, _E), jnp.float32),
        pltpu.SemaphoreType.DMA,
        pltpu.SemaphoreType.DMA,
    ]
    fn = pl.kernel(_emb_body, out_type=out_type, mesh=mesh,
                   scratch_types=scratch,
                   compiler_params=pltpu.CompilerParams(
                       use_tc_tiling_on_sc=False))
    return fn(src_idx2d, tgt_idx2d, src_table, tgt_table)


def kernel(src_tokens, tgt_tokens, src_table, tgt_table):
    src_idx2d = src_tokens.astype(jnp.int32).reshape(_NW * _CH, _C)
    tgt_idx2d = tgt_tokens.astype(jnp.int32).reshape(_NW * _CH, _C)
    outs = _emb(src_idx2d, tgt_idx2d, src_table, tgt_table)
    return (outs[0].reshape(_B, _L, _E), outs[1].reshape(_B, _L, _E))


# speedup vs baseline: 1.7214x; 1.7214x over previous
import jax
import jax.numpy as jnp
from jax import lax
from jax.experimental import pallas as pl
from jax.experimental.pallas import tpu as pltpu
from jax.experimental.pallas import tpu_sc as plsc

_NC = 2
_NS = 16
_NW = _NC * _NS

_B = 4096
_L = 50
_E = 64
_T = _B * _L
_CS = 128
_CH = _T // (_NW * _CS)


def _emb_body(src_idx, tgt_idx, src_tab, tgt_tab, src_out, tgt_out,
              idxs, idxt, s0, s1):
    wid = lax.axis_index("s") * _NC + lax.axis_index("c")
    r0 = wid * (_CH * _CS)

    pltpu.sync_copy(src_idx.at[wid], idxs)
    pltpu.sync_copy(tgt_idx.at[wid], idxt)

    def scoped(buf0, buf1):
        def run_table(tab, out, idxv):
            def gather(i, buf, sem):
                pltpu.async_copy(tab.at[idxv.at[i]], buf, sem)

            def writeback(i, buf, sem):
                pltpu.make_async_copy(tab.at[idxv.at[i]], buf, sem).wait()
                pltpu.sync_copy(buf, out.at[pl.ds(r0 + i * _CS, _CS)])

            gather(0, buf0, s0)

            @pl.loop(0, _CH, step=2)
            def _pair(i):
                gather(i + 1, buf1, s1)
                writeback(i, buf0, s0)

                @pl.when(i + 2 < _CH)
                def _():
                    gather(i + 2, buf0, s0)

                writeback(i + 1, buf1, s1)

        run_table(src_tab, src_out, idxs)
        run_table(tgt_tab, tgt_out, idxt)

    pl.run_scoped(
        scoped,
        pltpu.VMEM((_CS, _E), jnp.float32),
        pltpu.VMEM((_CS, _E), jnp.float32),
    )


@jax.jit
def _emb(src_idx, tgt_idx, src_table, tgt_table):
    mesh = plsc.VectorSubcoreMesh(core_axis_name="c", subcore_axis_name="s")
    out_type = (jax.ShapeDtypeStruct((_T, _E), jnp.float32),
                jax.ShapeDtypeStruct((_T, _E), jnp.float32))
    scratch = [
        pltpu.VMEM((_CH, _CS), jnp.int32),
        pltpu.VMEM((_CH, _CS), jnp.int32),
        pltpu.SemaphoreType.DMA,
        pltpu.SemaphoreType.DMA,
    ]
    fn = pl.kernel(
        _emb_body, out_type=out_type, mesh=mesh, scratch_types=scratch,
        compiler_params=pltpu.CompilerParams(use_tc_tiling_on_sc=False),
    )
    src_flat, tgt_flat = fn(src_idx, tgt_idx, src_table, tgt_table)
    return (src_flat.reshape(_B, _L, _E), tgt_flat.reshape(_B, _L, _E))


def kernel(src_tokens, tgt_tokens, src_table, tgt_table):
    return _emb(src_tokens.astype(jnp.int32).reshape(_NW, _CH, _CS),
                tgt_tokens.astype(jnp.int32).reshape(_NW, _CH, _CS),
                src_table, tgt_table)
